# baseline (speedup 1.0000x reference)
import functools

import jax
import jax.numpy as jnp
from jax import lax
from jax.experimental import pallas as pl
from jax.experimental.pallas import tpu as pltpu
from jax.experimental.pallas import tpu_sc as plsc

_NC = 2
_NS = 16
_NW = _NC * _NS

_T = 128
_PCHUNK = 640
_TPJ = _PCHUNK // _T


def _sc_gather_body(n_chunks, x_hbm, table_hbm, out_hbm, idx_v, rows_v, sem):
    wid = lax.axis_index("s") * _NC + lax.axis_index("c")
    base = wid * (n_chunks * _PCHUNK)

    def chunk_body(g, carry):
        p0 = base + g * _PCHUNK
        pltpu.sync_copy(x_hbm.at[pl.ds(p0 * 4, _PCHUNK * 4)], idx_v)
        copies = [
            pltpu.async_copy(
                table_hbm.at[idx_v.at[pl.ds(k * _T, _T)]],
                rows_v.at[pl.ds(k * _T, _T)],
                sem,
            )
            for k in range(4 * _TPJ)
        ]
        for c in copies:
            c.wait()
        for j in range(4):
            pltpu.sync_copy(
                rows_v.at[pl.ds(j * _PCHUNK, _PCHUNK)],
                out_hbm.at[pl.ds(p0, _PCHUNK), pl.ds(32 * j, 32)],
            )
        return carry

    lax.fori_loop(0, n_chunks, chunk_body, 0)


def _sc_gather(x_flat, table):
    n = x_flat.shape[0]
    d = table.shape[1]
    pn = n // 4
    assert pn % (_NW * _PCHUNK) == 0
    n_chunks = pn // (_NW * _PCHUNK)
    mesh = plsc.VectorSubcoreMesh(
        core_axis_name="c", subcore_axis_name="s",
        num_cores=_NC, num_subcores=_NS,
    )
    kern = pl.kernel(
        functools.partial(_sc_gather_body, n_chunks),
        out_type=jax.ShapeDtypeStruct((pn, 4 * d), jnp.float32),
        mesh=mesh,
        scratch_types=[
            pltpu.VMEM((4 * _PCHUNK,), jnp.int32),
            pltpu.VMEM((4 * _PCHUNK, d), jnp.float32),
            pltpu.SemaphoreType.DMA,
        ],
        compiler_params=pltpu.CompilerParams(use_tc_tiling_on_sc=False),
    )
    return kern(x_flat, table)


_BLK = 4096

_RC = 1600
_RCHUNKS = 625
_RPW = 20


def _sc_repack_body(tt_hbm, out_hbm, in_v, out_v, iota16):
    wid = lax.axis_index("s") * _NC + lax.axis_index("c")

    def chunk_body(k, carry):
        c = wid + _NW * k

        @pl.when(c < _RCHUNKS)
        def _():
            v0 = c * _RC
            pltpu.sync_copy(tt_hbm.at[:, pl.ds(v0, _RC)], in_v)
            for e in range(32):
                col = jnp.full((16,), e, jnp.int32)

                def lg_body(lg, carry2):
                    vals = in_v[e, pl.ds(lg * 16, 16)]
                    plsc.store_scatter(
                        out_v, [lg * 16 + iota16, col], vals
                    )
                    return carry2

                lax.fori_loop(0, _RC // 16, lg_body, 0)
            pltpu.sync_copy(out_v, out_hbm.at[pl.ds(v0, _RC)])

        return carry

    lax.fori_loop(0, _RPW, chunk_body, 0)


def _sc_repack(table_t):
    v = table_t.shape[1]
    mesh = plsc.VectorSubcoreMesh(
        core_axis_name="c", subcore_axis_name="s",
        num_cores=_NC, num_subcores=_NS,
    )
    def body(tt_hbm, out_hbm, in_v, out_v):
        iota16 = lax.iota(jnp.int32, 16)
        _sc_repack_body(tt_hbm, out_hbm, in_v, out_v, iota16)

    kern = pl.kernel(
        body,
        out_type=jax.ShapeDtypeStruct((v, 32), jnp.float32),
        mesh=mesh,
        scratch_types=[
            pltpu.VMEM((32, _RC), jnp.float32),
            pltpu.VMEM((_RC, 32), jnp.float32),
        ],
        compiler_params=pltpu.CompilerParams(
            use_tc_tiling_on_sc=False, needs_layout_passes=False,
        ),
    )
    return kern(table_t)


def _proj_body(ep_ref, d1_ref, d2_ref, o1_ref, o2_ref):
    e = ep_ref[...]
    o1_ref[...] = lax.dot_general(
        e, d1_ref[...], (((1,), (0,)), ((), ())),
        preferred_element_type=jnp.float32,
        precision=lax.Precision.HIGHEST,
    )
    o2_ref[...] = lax.dot_general(
        e, d2_ref[...], (((1,), (0,)), ((), ())),
        preferred_element_type=jnp.float32,
        precision=lax.Precision.HIGHEST,
    )


def _tc_project(emb_p, d1, d2):
    pn = emb_p.shape[0]
    grid = (pn // _BLK,)
    out1, out2 = pl.pallas_call(
        _proj_body,
        grid=grid,
        in_specs=[
            pl.BlockSpec((_BLK, 128), lambda i: (i, 0)),
            pl.BlockSpec((128, 128), lambda i: (0, 0)),
            pl.BlockSpec((128, 128), lambda i: (0, 0)),
        ],
        out_specs=[
            pl.BlockSpec((_BLK, 128), lambda i: (i, 0)),
            pl.BlockSpec((_BLK, 128), lambda i: (i, 0)),
        ],
        out_shape=[
            jax.ShapeDtypeStruct((pn, 128), jnp.float32),
            jax.ShapeDtypeStruct((pn, 128), jnp.float32),
        ],
        compiler_params=pltpu.CompilerParams(
            dimension_semantics=("arbitrary",),
        ),
    )(emb_p, d1, d2)
    return out1, out2


def kernel(x, table, mat, mat1):
    batch, length = x.shape
    xp = (
        x.reshape(batch * length // (4 * _PCHUNK), _PCHUNK, 4)
        .transpose(0, 2, 1)
        .reshape(-1)
    )
    table_lin = _sc_repack(jnp.transpose(table))
    emb_p = _sc_gather(xp, table_lin)
    d1 = jnp.kron(jnp.eye(4, dtype=mat.dtype), mat.T)
    d2 = jnp.kron(jnp.eye(4, dtype=mat1.dtype), mat1.T)
    op1, op2 = _tc_project(emb_p, d1, d2)
    shp = (batch, length, mat.shape[0])
    return (op1.reshape(shp), op2.reshape(shp))


# speedup vs baseline: 2.8140x; 2.8140x over previous
import functools

import jax
import jax.numpy as jnp
from jax import lax
from jax.experimental import pallas as pl
from jax.experimental.pallas import tpu as pltpu
from jax.experimental.pallas import tpu_sc as plsc

_NC = 2
_NS = 16
_NW = _NC * _NS

_T = 128
_PCHUNK = 640
_TPJ = _PCHUNK // _T


def _sc_gather_body(n_chunks, x_hbm, table_hbm, out_hbm, idx_v, rows_v, sem):
    wid = lax.axis_index("s") * _NC + lax.axis_index("c")
    base = wid * (n_chunks * _PCHUNK)

    def chunk_body(g, carry):
        p0 = base + g * _PCHUNK
        pltpu.sync_copy(x_hbm.at[pl.ds(p0 * 4, _PCHUNK * 4)], idx_v)
        copies = [
            pltpu.async_copy(
                table_hbm.at[idx_v.at[pl.ds(k * _T, _T)]],
                rows_v.at[pl.ds(k * _T, _T)],
                sem,
            )
            for k in range(4 * _TPJ)
        ]
        for c in copies:
            c.wait()
        for j in range(4):
            pltpu.sync_copy(
                rows_v.at[pl.ds(j * _PCHUNK, _PCHUNK)],
                out_hbm.at[pl.ds(p0, _PCHUNK), pl.ds(32 * j, 32)],
            )
        return carry

    lax.fori_loop(0, n_chunks, chunk_body, 0)


def _sc_gather(x_flat, table):
    n = x_flat.shape[0]
    d = table.shape[1]
    pn = n // 4
    assert pn % (_NW * _PCHUNK) == 0
    n_chunks = pn // (_NW * _PCHUNK)
    mesh = plsc.VectorSubcoreMesh(
        core_axis_name="c", subcore_axis_name="s",
        num_cores=_NC, num_subcores=_NS,
    )
    kern = pl.kernel(
        functools.partial(_sc_gather_body, n_chunks),
        out_type=jax.ShapeDtypeStruct((pn, 4 * d), jnp.float32),
        mesh=mesh,
        scratch_types=[
            pltpu.VMEM((4 * _PCHUNK,), jnp.int32),
            pltpu.VMEM((4 * _PCHUNK, d), jnp.float32),
            pltpu.SemaphoreType.DMA,
        ],
        compiler_params=pltpu.CompilerParams(use_tc_tiling_on_sc=False),
    )
    return kern(x_flat, table)


_BLK = 4096

def _proj_body(ep_ref, d1_ref, d2_ref, o1_ref, o2_ref):
    e = ep_ref[...]
    o1_ref[...] = lax.dot_general(
        e, d1_ref[...], (((1,), (0,)), ((), ())),
        preferred_element_type=jnp.float32,
        precision=lax.Precision.HIGHEST,
    )
    o2_ref[...] = lax.dot_general(
        e, d2_ref[...], (((1,), (0,)), ((), ())),
        preferred_element_type=jnp.float32,
        precision=lax.Precision.HIGHEST,
    )


def _tc_project(emb_p, d1, d2):
    pn = emb_p.shape[0]
    grid = (pn // _BLK,)
    out1, out2 = pl.pallas_call(
        _proj_body,
        grid=grid,
        in_specs=[
            pl.BlockSpec((_BLK, 128), lambda i: (i, 0)),
            pl.BlockSpec((128, 128), lambda i: (0, 0)),
            pl.BlockSpec((128, 128), lambda i: (0, 0)),
        ],
        out_specs=[
            pl.BlockSpec((_BLK, 128), lambda i: (i, 0)),
            pl.BlockSpec((_BLK, 128), lambda i: (i, 0)),
        ],
        out_shape=[
            jax.ShapeDtypeStruct((pn, 128), jnp.float32),
            jax.ShapeDtypeStruct((pn, 128), jnp.float32),
        ],
        compiler_params=pltpu.CompilerParams(
            dimension_semantics=("parallel",),
        ),
    )(emb_p, d1, d2)
    return out1, out2


def kernel(x, table, mat, mat1):
    batch, length = x.shape
    xp = (
        x.reshape(batch * length // (4 * _PCHUNK), _PCHUNK, 4)
        .transpose(0, 2, 1)
        .reshape(-1)
    )
    emb_p = _sc_gather(xp, table)
    d1 = jnp.kron(jnp.eye(4, dtype=mat.dtype), mat.T)
    d2 = jnp.kron(jnp.eye(4, dtype=mat1.dtype), mat1.T)
    op1, op2 = _tc_project(emb_p, d1, d2)
    shp = (batch, length, mat.shape[0])
    return (op1.reshape(shp), op2.reshape(shp))


# speedup vs baseline: 2.8153x; 1.0005x over previous
import functools

import jax
import jax.numpy as jnp
from jax import lax
from jax.experimental import pallas as pl
from jax.experimental.pallas import tpu as pltpu
from jax.experimental.pallas import tpu_sc as plsc

_NC = 2
_NS = 16
_NW = _NC * _NS

_T = 128
_PCHUNK = 640
_TPJ = _PCHUNK // _T


def _sc_gather_body(n_chunks, x_hbm, table_hbm, out_hbm, idx_v, rows_v, sem):
    wid = lax.axis_index("s") * _NC + lax.axis_index("c")
    base = wid * (n_chunks * _PCHUNK)

    def chunk_body(g, carry):
        p0 = base + g * _PCHUNK
        pltpu.sync_copy(x_hbm.at[pl.ds(p0 * 4, _PCHUNK * 4)], idx_v)
        copies = [
            pltpu.async_copy(
                table_hbm.at[idx_v.at[pl.ds(k * _T, _T)]],
                rows_v.at[pl.ds(k * _T, _T)],
                sem,
            )
            for k in range(4 * _TPJ)
        ]
        for c in copies:
            c.wait()
        for j in range(4):
            pltpu.sync_copy(
                rows_v.at[pl.ds(j * _PCHUNK, _PCHUNK)],
                out_hbm.at[pl.ds(p0, _PCHUNK), pl.ds(32 * j, 32)],
            )
        return carry

    lax.fori_loop(0, n_chunks, chunk_body, 0)


def _sc_gather(x_flat, table):
    n = x_flat.shape[0]
    d = table.shape[1]
    pn = n // 4
    assert pn % (_NW * _PCHUNK) == 0
    n_chunks = pn // (_NW * _PCHUNK)
    mesh = plsc.VectorSubcoreMesh(
        core_axis_name="c", subcore_axis_name="s",
        num_cores=_NC, num_subcores=_NS,
    )
    kern = pl.kernel(
        functools.partial(_sc_gather_body, n_chunks),
        out_type=jax.ShapeDtypeStruct((pn, 4 * d), jnp.float32),
        mesh=mesh,
        scratch_types=[
            pltpu.VMEM((4 * _PCHUNK,), jnp.int32),
            pltpu.VMEM((4 * _PCHUNK, d), jnp.float32),
            pltpu.SemaphoreType.DMA,
        ],
        compiler_params=pltpu.CompilerParams(use_tc_tiling_on_sc=False),
    )
    return kern(x_flat, table)


_BLK = 8192

def _proj_body(ep_ref, d1_ref, d2_ref, o1_ref, o2_ref):
    e = ep_ref[...]
    o1_ref[...] = lax.dot_general(
        e, d1_ref[...], (((1,), (0,)), ((), ())),
        preferred_element_type=jnp.float32,
        precision=lax.Precision.HIGHEST,
    )
    o2_ref[...] = lax.dot_general(
        e, d2_ref[...], (((1,), (0,)), ((), ())),
        preferred_element_type=jnp.float32,
        precision=lax.Precision.HIGHEST,
    )


def _tc_project(emb_p, d1, d2):
    pn = emb_p.shape[0]
    grid = (pn // _BLK,)
    out1, out2 = pl.pallas_call(
        _proj_body,
        grid=grid,
        in_specs=[
            pl.BlockSpec((_BLK, 128), lambda i: (i, 0)),
            pl.BlockSpec((128, 128), lambda i: (0, 0)),
            pl.BlockSpec((128, 128), lambda i: (0, 0)),
        ],
        out_specs=[
            pl.BlockSpec((_BLK, 128), lambda i: (i, 0)),
            pl.BlockSpec((_BLK, 128), lambda i: (i, 0)),
        ],
        out_shape=[
            jax.ShapeDtypeStruct((pn, 128), jnp.float32),
            jax.ShapeDtypeStruct((pn, 128), jnp.float32),
        ],
        compiler_params=pltpu.CompilerParams(
            dimension_semantics=("parallel",),
        ),
    )(emb_p, d1, d2)
    return out1, out2


def kernel(x, table, mat, mat1):
    batch, length = x.shape
    xp = (
        x.reshape(batch * length // (4 * _PCHUNK), _PCHUNK, 4)
        .transpose(0, 2, 1)
        .reshape(-1)
    )
    emb_p = _sc_gather(xp, table)
    d1 = jnp.kron(jnp.eye(4, dtype=mat.dtype), mat.T)
    d2 = jnp.kron(jnp.eye(4, dtype=mat1.dtype), mat1.T)
    op1, op2 = _tc_project(emb_p, d1, d2)
    shp = (batch, length, mat.shape[0])
    return (op1.reshape(shp), op2.reshape(shp))
